# baseline (device time: 8997 ns/iter reference)
import jax
import jax.numpy as jnp
from jax import lax
from jax.experimental import pallas as pl
from jax.experimental.pallas import tpu as pltpu

_NBLK = 4


def kernel(x):
    m_per, n_per = x.shape
    bm = m_per // _NBLK
    rows = bm // 128

    def body(x_hbm, out_ref, buf, partial_ref, peer_ref,
             load_sems, send_sems, recv_sems):
        my_x = lax.axis_index("x")
        my_y = lax.axis_index("y")
        peer = (my_x, 1 - my_y)
        barrier_sem = pltpu.get_barrier_semaphore()

        def load(i):
            return pltpu.make_async_copy(
                x_hbm.at[pl.ds(i * bm, bm), :],
                buf.at[i % 2],
                load_sems.at[i % 2],
            )

        def block_rdma(k):
            return pltpu.make_async_remote_copy(
                src_ref=partial_ref.at[pl.ds(k * rows, rows)],
                dst_ref=peer_ref.at[pl.ds(k * rows, rows)],
                send_sem=send_sems.at[k],
                recv_sem=recv_sems.at[k],
                device_id=peer,
                device_id_type=pl.DeviceIdType.MESH,
            )

        pl.semaphore_signal(
            barrier_sem, inc=1, device_id=peer,
            device_id_type=pl.DeviceIdType.MESH,
        )

        load(0).start()
        for i in range(_NBLK):
            if i + 1 < _NBLK:
                load(i + 1).start()
            load(i).wait()
            s = jnp.sum(buf[i % 2].astype(jnp.float32), axis=1)
            partial_ref[pl.ds(i * rows, rows), :] = s.reshape(rows, 128)
            if i == 0:
                pl.semaphore_wait(barrier_sem, 1)
            block_rdma(i).start()

        for k in range(_NBLK):
            rdma = block_rdma(k)
            rdma.wait_send()
            rdma.wait_recv()
        out_ref[:, :] = partial_ref[:, :] + peer_ref[:, :]

    out = pl.pallas_call(
        body,
        out_shape=jax.ShapeDtypeStruct((m_per // 128, 128), jnp.float32),
        in_specs=[pl.BlockSpec(memory_space=pl.ANY)],
        out_specs=pl.BlockSpec(memory_space=pltpu.VMEM),
        scratch_shapes=[
            pltpu.VMEM((2, bm, n_per), x.dtype),
            pltpu.VMEM((m_per // 128, 128), jnp.float32),
            pltpu.VMEM((m_per // 128, 128), jnp.float32),
            pltpu.SemaphoreType.DMA((2,)),
            pltpu.SemaphoreType.DMA((_NBLK,)),
            pltpu.SemaphoreType.DMA((_NBLK,)),
        ],
        compiler_params=pltpu.CompilerParams(collective_id=0),
    )(x)
    return out.reshape(m_per, 1)
